# baseline (speedup 1.0000x reference)
import functools

import jax
import jax.numpy as jnp
from jax import lax
from jax.experimental import pallas as pl
from jax.experimental.pallas import tpu as pltpu
from jax.experimental.pallas import tpu_sc as plsc

V = 10000
B = 4
HD = 128
E = 320000

VP = 10240
NC = 2
NS = 16
NW = NC * NS
CH = 128
KPT = 80
EP = NW * KPT * CH
RPT = VP // NS
NSTR = 4

BLK = 256
GRID = VP // BLK


def _deg_body(dst_hbm, ones_hbm, zeros_hbm, out_hbm, dst_v, ones_v, acc):
    c = lax.axis_index("c")
    s = lax.axis_index("s")
    wid = c * NS + s
    pltpu.sync_copy(dst_hbm.at[wid], dst_v)
    pltpu.sync_copy(ones_hbm, ones_v)
    pltpu.sync_copy(zeros_hbm.at[pl.ds(s * RPT, RPT)], acc.at[pl.ds(s * RPT, RPT)])
    plsc.subcore_barrier()

    def body(j, carry):
        pltpu.sync_copy(ones_v, acc.at[dst_v.at[j]], add=True)
        return carry

    lax.fori_loop(0, KPT, body, 0)
    plsc.subcore_barrier()
    pltpu.sync_copy(acc.at[pl.ds(s * RPT, RPT)], out_hbm.at[c, pl.ds(s * RPT, RPT)])


@functools.cache
def _sc_calls():
    mesh = plsc.VectorSubcoreMesh(core_axis_name="c", subcore_axis_name="s")
    deg_call = pl.kernel(
        _deg_body,
        out_type=jax.ShapeDtypeStruct((NC, VP, HD), jnp.float32),
        mesh=mesh,
        scratch_types=[
            pltpu.VMEM((KPT, CH), jnp.int32),
            pltpu.VMEM((CH, HD), jnp.float32),
            pltpu.VMEM_SHARED((VP, HD), jnp.float32),
        ],
    )
    agg_call = pl.kernel(
        _agg_body,
        out_type=jax.ShapeDtypeStruct((NC, VP, HD), jnp.float32),
        mesh=mesh,
        scratch_types=[
            pltpu.VMEM((2, 2, CH), jnp.int32),
            pltpu.VMEM((2, CH, HD), jnp.float32),
            pltpu.VMEM_SHARED((VP, HD), jnp.float32),
            pltpu.SemaphoreType.DMA,
            pltpu.SemaphoreType.DMA,
            pltpu.SemaphoreType.DMA,
            pltpu.SemaphoreType.DMA,
        ],
    )
    return deg_call, agg_call


def _agg_body(ys_hbm, src_hbm, dst_hbm, zeros_hbm, out_hbm,
              idx_v, rows, acc, g0, g1, i0, i1):
    c = lax.axis_index("c")
    s = lax.axis_index("s")
    wid = c * NS + s
    gs = (g0, g1)
    isems = (i0, i1)

    def fetch_idx(j, b):
        pltpu.async_copy(src_hbm.at[wid, j], idx_v.at[b, 0], isems[b])
        pltpu.async_copy(dst_hbm.at[wid, j], idx_v.at[b, 1], isems[b])

    def wait_idx(j, b):
        pltpu.make_async_copy(src_hbm.at[wid, j], idx_v.at[b, 0], isems[b]).wait()
        pltpu.make_async_copy(dst_hbm.at[wid, j], idx_v.at[b, 1], isems[b]).wait()

    def gather(b):
        for k in range(NSTR):
            pltpu.async_copy(
                ys_hbm.at[idx_v.at[b, 0, pl.ds(k * (CH // NSTR), CH // NSTR)]],
                rows.at[b, pl.ds(k * (CH // NSTR), CH // NSTR)], gs[b])

    def wait_gather(b):
        for k in range(NSTR):
            pltpu.make_async_copy(
                ys_hbm.at[idx_v.at[b, 0, pl.ds(k * (CH // NSTR), CH // NSTR)]],
                rows.at[b, pl.ds(k * (CH // NSTR), CH // NSTR)], gs[b]).wait()

    fetch_idx(0, 0)
    fetch_idx(1, 1)
    pltpu.sync_copy(zeros_hbm.at[pl.ds(s * RPT, RPT)], acc.at[pl.ds(s * RPT, RPT)])
    plsc.subcore_barrier()
    wait_idx(0, 0)
    gather(0)

    def outer(i, carry):
        for b in range(2):
            j = 2 * i + b
            nb = 1 - b
            wait_gather(b)

            @pl.when(j + 1 < KPT)
            def _():
                wait_idx(j + 1, nb)
                gather(nb)

            pltpu.sync_copy(rows.at[b], acc.at[idx_v.at[b, 1]], add=True)

            @pl.when(j + 2 < KPT)
            def _():
                fetch_idx(j + 2, b)
        return carry

    lax.fori_loop(0, KPT // 2, outer, 0)
    plsc.subcore_barrier()
    pltpu.sync_copy(acc.at[pl.ds(s * RPT, RPT)], out_hbm.at[c, pl.ds(s * RPT, RPT)])


def _prep_body(lat_ref, wl_ref, b_ref, out_ref):
    out_ref[...] = (
        jnp.dot(lat_ref[...], wl_ref[...], preferred_element_type=jnp.float32, precision=lax.Precision.HIGHEST)
        + b_ref[...]
    )


def _dis_from_partials(p0_ref, p1_ref):
    p = p0_ref[...][:, 0:1] + p1_ref[...][:, 0:1]
    deg = 4.0 * p + 1.0
    return lax.rsqrt(deg)


def _in_body(xyz_ref, w3_ref, co_ref, p0_ref, p1_ref, w_ref, xw_ref, ys_ref):
    base = jnp.dot(xyz_ref[...], w3_ref[...], preferred_element_type=jnp.float32, precision=lax.Precision.HIGHEST)
    co = co_ref[...]
    W = w_ref[...]
    dis = _dis_from_partials(p0_ref, p1_ref)
    for b in range(B):
        h = jnp.maximum(base + co[b:b + 1, :], 0.0)
        xw = jnp.dot(h, W, preferred_element_type=jnp.float32, precision=lax.Precision.HIGHEST)
        xw_ref[b, :, :] = xw
        if b == 0:
            ys_ref[...] = dis * xw


def _mid_body(xw_ref, acc_ref, p0_ref, p1_ref, b_ref, wn_ref, xwn_ref, ys_ref):
    dis = _dis_from_partials(p0_ref, p1_ref)
    bias = b_ref[...][0:1, :]
    Wn = wn_ref[...]
    a = acc_ref[0, :, :] + acc_ref[1, :, :]
    g0 = jnp.maximum(4.0 * dis * a + (dis * dis) * xw_ref[0, :, :] + bias, 0.0)
    xwn0 = jnp.dot(g0, Wn, preferred_element_type=jnp.float32, precision=lax.Precision.HIGHEST)
    xwn_ref[0, :, :] = xwn0
    ys_ref[...] = dis * xwn0
    for b in range(1, B):
        g = jnp.maximum(xw_ref[b, :, :] + bias, 0.0)
        xwn_ref[b, :, :] = jnp.dot(g, Wn, preferred_element_type=jnp.float32, precision=lax.Precision.HIGHEST)


def _out_body(xw_ref, acc_ref, p0_ref, p1_ref, b_ref, wo_ref, bo_ref, o_ref):
    dis = _dis_from_partials(p0_ref, p1_ref)
    bias = b_ref[...][0:1, :]
    Wo = wo_ref[...]
    bo = bo_ref[...][0:1, :]
    a = acc_ref[0, :, :] + acc_ref[1, :, :]
    g0 = jnp.maximum(4.0 * dis * a + (dis * dis) * xw_ref[0, :, :] + bias, 0.0)
    o_ref[0, :, :] = jnp.dot(g0, Wo, preferred_element_type=jnp.float32, precision=lax.Precision.HIGHEST) + bo
    for b in range(1, B):
        g = jnp.maximum(xw_ref[b, :, :] + bias, 0.0)
        o_ref[b, :, :] = jnp.dot(g, Wo, preferred_element_type=jnp.float32, precision=lax.Precision.HIGHEST) + bo


def _row_spec(minor):
    return pl.BlockSpec((BLK, minor), lambda i: (i, 0))


def _full_spec(shape):
    nd = len(shape)
    return pl.BlockSpec(shape, lambda i, _nd=nd: (0,) * _nd)


_in_call = pl.pallas_call(
    _in_body,
    grid=(GRID,),
    in_specs=[
        _row_spec(128),
        _full_spec((128, 128)),
        _full_spec((8, 128)),
        _row_spec(128),
        _row_spec(128),
        _full_spec((128, 128)),
    ],
    out_specs=[
        pl.BlockSpec((B, BLK, 128), lambda i: (0, i, 0)),
        _row_spec(128),
    ],
    out_shape=[
        jax.ShapeDtypeStruct((B, VP, 128), jnp.float32),
        jax.ShapeDtypeStruct((VP, 128), jnp.float32),
    ],
)

_mid_call = pl.pallas_call(
    _mid_body,
    grid=(GRID,),
    in_specs=[
        pl.BlockSpec((B, BLK, 128), lambda i: (0, i, 0)),
        pl.BlockSpec((NC, BLK, 128), lambda i: (0, i, 0)),
        _row_spec(128),
        _row_spec(128),
        _full_spec((8, 128)),
        _full_spec((128, 128)),
    ],
    out_specs=[
        pl.BlockSpec((B, BLK, 128), lambda i: (0, i, 0)),
        _row_spec(128),
    ],
    out_shape=[
        jax.ShapeDtypeStruct((B, VP, 128), jnp.float32),
        jax.ShapeDtypeStruct((VP, 128), jnp.float32),
    ],
)

_fin_call = pl.pallas_call(
    _out_body,
    grid=(GRID,),
    in_specs=[
        pl.BlockSpec((B, BLK, 128), lambda i: (0, i, 0)),
        pl.BlockSpec((NC, BLK, 128), lambda i: (0, i, 0)),
        _row_spec(128),
        _row_spec(128),
        _full_spec((8, 128)),
        _full_spec((128, 128)),
        _full_spec((8, 128)),
    ],
    out_specs=pl.BlockSpec((B, BLK, 128), lambda i: (0, i, 0)),
    out_shape=jax.ShapeDtypeStruct((B, VP, 128), jnp.float32),
)

_prep_call = pl.pallas_call(
    _prep_body,
    out_shape=jax.ShapeDtypeStruct((8, 128), jnp.float32),
)


@jax.jit
def kernel(vertex_xyz, latent, edge_index, W_in, b_in,
           W_c0, b_c0, W_c1, b_c1, W_c2, b_c2, W_out, b_out):
    f32 = jnp.float32

    src = edge_index[0]
    dst = edge_index[1]
    pad = jnp.full((EP - E,), V, jnp.int32)
    src3 = jnp.concatenate([src, pad]).reshape(NW, KPT, CH)
    dst3 = jnp.concatenate([dst, pad]).reshape(NW, KPT, CH)

    xyz128 = jnp.zeros((VP, 128), f32).at[:V, :3].set(vertex_xyz)
    W3p = jnp.zeros((128, 128), f32).at[:3, :].set(W_in[:3])
    Wl = W_in[3:]
    lat8 = jnp.zeros((8, Wl.shape[0]), f32).at[:B, :].set(latent)
    bin8 = jnp.broadcast_to(b_in[None, :], (8, 128))
    bc = [jnp.broadcast_to(bb[None, :], (8, 128)) for bb in (b_c0, b_c1, b_c2)]
    Wo128 = jnp.zeros((128, 128), f32).at[:, :3].set(W_out)
    bo8 = jnp.zeros((8, 128), f32).at[0, :3].set(b_out)

    ones128 = jnp.ones((CH, HD), f32)
    zeros128 = jnp.zeros((VP, HD), f32)

    _deg_call, _agg_call = _sc_calls()
    degp = _deg_call(dst3, ones128, zeros128)
    p0 = degp[0]
    p1 = degp[1]

    consts = _prep_call(lat8, Wl, bin8)
    xw, ys = _in_call(xyz128, W3p, consts, p0, p1, W_c0)

    for l, Wn in ((0, W_c1), (1, W_c2)):
        accp = _agg_call(ys, src3, dst3, zeros128)
        xw, ys = _mid_call(xw, accp, p0, p1, bc[l], Wn)

    accp = _agg_call(ys, src3, dst3, zeros128)
    o = _fin_call(xw, accp, p0, p1, bc[2], Wo128, bo8)
    return o[:, :V, :3]


# speedup vs baseline: 2.5241x; 2.5241x over previous
import functools

import jax
import jax.numpy as jnp
from jax import lax
from jax.experimental import pallas as pl
from jax.experimental.pallas import tpu as pltpu
from jax.experimental.pallas import tpu_sc as plsc

V = 10000
B = 4
HD = 128
E = 320000

VP = 10240
NC = 2
NS = 16
NW = NC * NS
CH = 128
KPT = 80
EP = NW * KPT * CH
RPT = VP // NS
NSTR = 4

BLK = 256
GRID = VP // BLK


def _deg_body(dst_hbm, ones_hbm, zeros_hbm, out_hbm, dst_v, ones_v, acc):
    c = lax.axis_index("c")
    s = lax.axis_index("s")
    wid = c * NS + s
    pltpu.sync_copy(dst_hbm.at[wid], dst_v)
    pltpu.sync_copy(ones_hbm, ones_v)
    pltpu.sync_copy(zeros_hbm.at[pl.ds(s * RPT, RPT)], acc.at[pl.ds(s * RPT, RPT)])
    plsc.subcore_barrier()

    def body(j, carry):
        pltpu.sync_copy(ones_v, acc.at[dst_v.at[j]], add=True)
        return carry

    lax.fori_loop(0, KPT, body, 0)
    plsc.subcore_barrier()
    pltpu.sync_copy(acc.at[pl.ds(s * RPT, RPT)], out_hbm.at[c, pl.ds(s * RPT, RPT)])


@functools.cache
def _sc_calls():
    mesh = plsc.VectorSubcoreMesh(core_axis_name="c", subcore_axis_name="s")
    deg_call = pl.kernel(
        _deg_body,
        out_type=jax.ShapeDtypeStruct((NC, VP, HD), jnp.float32),
        mesh=mesh,
        scratch_types=[
            pltpu.VMEM((KPT, CH), jnp.int32),
            pltpu.VMEM((CH, HD), jnp.float32),
            pltpu.VMEM_SHARED((VP, HD), jnp.float32),
        ],
    )
    agg_call = pl.kernel(
        _agg_body,
        out_type=jax.ShapeDtypeStruct((NC, VP, HD), jnp.float32),
        mesh=mesh,
        scratch_types=[
            pltpu.VMEM((2, 2, CH), jnp.int32),
            pltpu.VMEM((2, CH, HD), jnp.float32),
            pltpu.VMEM_SHARED((VP, HD), jnp.float32),
            pltpu.SemaphoreType.DMA,
            pltpu.SemaphoreType.DMA,
            pltpu.SemaphoreType.DMA,
            pltpu.SemaphoreType.DMA,
        ],
    )
    return deg_call, agg_call


def _agg_body(ys_hbm, src_hbm, dst_hbm, zeros_hbm, out_hbm,
              idx_v, rows, acc, g0, g1, i0, i1):
    c = lax.axis_index("c")
    s = lax.axis_index("s")
    wid = c * NS + s
    gs = (g0, g1)
    isems = (i0, i1)

    def fetch_idx(j, b):
        pltpu.async_copy(src_hbm.at[wid, j], idx_v.at[b, 0], isems[b])
        pltpu.async_copy(dst_hbm.at[wid, j], idx_v.at[b, 1], isems[b])

    def wait_idx(j, b):
        pltpu.make_async_copy(src_hbm.at[wid, j], idx_v.at[b, 0], isems[b]).wait()
        pltpu.make_async_copy(dst_hbm.at[wid, j], idx_v.at[b, 1], isems[b]).wait()

    def gather(b):
        for k in range(NSTR):
            pltpu.async_copy(
                ys_hbm.at[idx_v.at[b, 0, pl.ds(k * (CH // NSTR), CH // NSTR)]],
                rows.at[b, pl.ds(k * (CH // NSTR), CH // NSTR)], gs[b])

    def wait_gather(b):
        for k in range(NSTR):
            pltpu.make_async_copy(
                ys_hbm.at[idx_v.at[b, 0, pl.ds(k * (CH // NSTR), CH // NSTR)]],
                rows.at[b, pl.ds(k * (CH // NSTR), CH // NSTR)], gs[b]).wait()

    fetch_idx(0, 0)
    fetch_idx(1, 1)
    pltpu.sync_copy(zeros_hbm.at[pl.ds(s * RPT, RPT)], acc.at[pl.ds(s * RPT, RPT)])
    plsc.subcore_barrier()
    wait_idx(0, 0)
    gather(0)

    def outer(i, carry):
        for b in range(2):
            j = 2 * i + b
            nb = 1 - b
            wait_gather(b)

            @pl.when(j + 1 < KPT)
            def _():
                wait_idx(j + 1, nb)
                gather(nb)

            pltpu.sync_copy(rows.at[b], acc.at[idx_v.at[b, 1]], add=True)

            @pl.when(j + 2 < KPT)
            def _():
                fetch_idx(j + 2, b)
        return carry

    lax.fori_loop(0, KPT // 2, outer, 0)
    plsc.subcore_barrier()
    pltpu.sync_copy(acc.at[pl.ds(s * RPT, RPT)], out_hbm.at[c, pl.ds(s * RPT, RPT)])


def _prep_body(lat_ref, wl_ref, b_ref, out_ref):
    out_ref[...] = (
        jnp.dot(lat_ref[...], wl_ref[...], preferred_element_type=jnp.float32, precision=lax.Precision.HIGHEST)
        + b_ref[...]
    )


def _dis_from_partials(p0_ref, p1_ref):
    p = p0_ref[...][:, 0:1] + p1_ref[...][:, 0:1]
    deg = 4.0 * p + 1.0
    return lax.rsqrt(deg)


def _in_body(xyz_ref, w3_ref, co_ref, p0_ref, p1_ref, w_ref, xw_ref, ys_ref):
    base = jnp.dot(xyz_ref[...], w3_ref[...], preferred_element_type=jnp.float32, precision=lax.Precision.HIGHEST)
    co = co_ref[...]
    W = w_ref[...]
    dis = _dis_from_partials(p0_ref, p1_ref)
    for b in range(B):
        h = jnp.maximum(base + co[b:b + 1, :], 0.0)
        xw = jnp.dot(h, W, preferred_element_type=jnp.float32, precision=lax.Precision.HIGHEST)
        xw_ref[b, :, :] = xw
        if b == 0:
            ys_ref[...] = dis * xw


def _mid_body(xw_ref, acc_ref, p0_ref, p1_ref, b_ref, wn_ref, xwn_ref, ys_ref):
    dis = _dis_from_partials(p0_ref, p1_ref)
    bias = b_ref[...][0:1, :]
    Wn = wn_ref[...]
    a = acc_ref[0, :, :] + acc_ref[1, :, :]
    g0 = jnp.maximum(4.0 * dis * a + (dis * dis) * xw_ref[0, :, :] + bias, 0.0)
    xwn0 = jnp.dot(g0, Wn, preferred_element_type=jnp.float32, precision=lax.Precision.HIGHEST)
    xwn_ref[0, :, :] = xwn0
    ys_ref[...] = dis * xwn0
    for b in range(1, B):
        g = jnp.maximum(xw_ref[b, :, :] + bias, 0.0)
        xwn_ref[b, :, :] = jnp.dot(g, Wn, preferred_element_type=jnp.float32, precision=lax.Precision.HIGHEST)


def _out_body(xw_ref, acc_ref, p0_ref, p1_ref, b_ref, wo_ref, bo_ref, o_ref):
    dis = _dis_from_partials(p0_ref, p1_ref)
    bias = b_ref[...][0:1, :]
    Wo = wo_ref[...]
    bo = bo_ref[...][0:1, :]
    a = acc_ref[0, :, :] + acc_ref[1, :, :]
    g0 = jnp.maximum(4.0 * dis * a + (dis * dis) * xw_ref[0, :, :] + bias, 0.0)
    o_ref[0, :, :] = jnp.dot(g0, Wo, preferred_element_type=jnp.float32, precision=lax.Precision.HIGHEST) + bo
    for b in range(1, B):
        g = jnp.maximum(xw_ref[b, :, :] + bias, 0.0)
        o_ref[b, :, :] = jnp.dot(g, Wo, preferred_element_type=jnp.float32, precision=lax.Precision.HIGHEST) + bo


def _row_spec(minor):
    return pl.BlockSpec((BLK, minor), lambda i: (i, 0))


def _full_spec(shape):
    nd = len(shape)
    return pl.BlockSpec(shape, lambda i, _nd=nd: (0,) * _nd)


_in_call = pl.pallas_call(
    _in_body,
    grid=(GRID,),
    in_specs=[
        _row_spec(128),
        _full_spec((128, 128)),
        _full_spec((8, 128)),
        _row_spec(128),
        _row_spec(128),
        _full_spec((128, 128)),
    ],
    out_specs=[
        pl.BlockSpec((B, BLK, 128), lambda i: (0, i, 0)),
        _row_spec(128),
    ],
    out_shape=[
        jax.ShapeDtypeStruct((B, VP, 128), jnp.float32),
        jax.ShapeDtypeStruct((VP, 128), jnp.float32),
    ],
)

_mid_call = pl.pallas_call(
    _mid_body,
    grid=(GRID,),
    in_specs=[
        pl.BlockSpec((B, BLK, 128), lambda i: (0, i, 0)),
        pl.BlockSpec((NC, BLK, 128), lambda i: (0, i, 0)),
        _row_spec(128),
        _row_spec(128),
        _full_spec((8, 128)),
        _full_spec((128, 128)),
    ],
    out_specs=[
        pl.BlockSpec((B, BLK, 128), lambda i: (0, i, 0)),
        _row_spec(128),
    ],
    out_shape=[
        jax.ShapeDtypeStruct((B, VP, 128), jnp.float32),
        jax.ShapeDtypeStruct((VP, 128), jnp.float32),
    ],
)

_fin_call = pl.pallas_call(
    _out_body,
    grid=(GRID,),
    in_specs=[
        pl.BlockSpec((B, BLK, 128), lambda i: (0, i, 0)),
        pl.BlockSpec((NC, BLK, 128), lambda i: (0, i, 0)),
        _row_spec(128),
        _row_spec(128),
        _full_spec((8, 128)),
        _full_spec((128, 128)),
        _full_spec((8, 128)),
    ],
    out_specs=pl.BlockSpec((B, BLK, 128), lambda i: (0, i, 0)),
    out_shape=jax.ShapeDtypeStruct((B, VP, 128), jnp.float32),
)

_prep_call = pl.pallas_call(
    _prep_body,
    out_shape=jax.ShapeDtypeStruct((8, 128), jnp.float32),
)


@jax.jit
def kernel(vertex_xyz, latent, edge_index, W_in, b_in,
           W_c0, b_c0, W_c1, b_c1, W_c2, b_c2, W_out, b_out):
    f32 = jnp.float32

    src = edge_index[0]
    dst = edge_index[1]
    pad = V + (jnp.arange(EP - E, dtype=jnp.int32) % (VP - V))
    src3 = jnp.concatenate([src, pad]).reshape(NW, KPT, CH)
    dst3 = jnp.concatenate([dst, pad]).reshape(NW, KPT, CH)

    xyz128 = jnp.zeros((VP, 128), f32).at[:V, :3].set(vertex_xyz)
    W3p = jnp.zeros((128, 128), f32).at[:3, :].set(W_in[:3])
    Wl = W_in[3:]
    lat8 = jnp.zeros((8, Wl.shape[0]), f32).at[:B, :].set(latent)
    bin8 = jnp.broadcast_to(b_in[None, :], (8, 128))
    bc = [jnp.broadcast_to(bb[None, :], (8, 128)) for bb in (b_c0, b_c1, b_c2)]
    Wo128 = jnp.zeros((128, 128), f32).at[:, :3].set(W_out)
    bo8 = jnp.zeros((8, 128), f32).at[0, :3].set(b_out)

    ones128 = jnp.ones((CH, HD), f32)
    zeros128 = jnp.zeros((VP, HD), f32)

    _deg_call, _agg_call = _sc_calls()
    degp = _deg_call(dst3, ones128, zeros128)
    p0 = degp[0]
    p1 = degp[1]

    consts = _prep_call(lat8, Wl, bin8)
    xw, ys = _in_call(xyz128, W3p, consts, p0, p1, W_c0)

    for l, Wn in ((0, W_c1), (1, W_c2)):
        accp = _agg_call(ys, src3, dst3, zeros128)
        xw, ys = _mid_call(xw, accp, p0, p1, bc[l], Wn)

    accp = _agg_call(ys, src3, dst3, zeros128)
    o = _fin_call(xw, accp, p0, p1, bc[2], Wo128, bo8)
    return o[:, :V, :3]


# speedup vs baseline: 2.5282x; 1.0016x over previous
import functools

import jax
import jax.numpy as jnp
from jax import lax
from jax.experimental import pallas as pl
from jax.experimental.pallas import tpu as pltpu
from jax.experimental.pallas import tpu_sc as plsc

V = 10000
B = 4
HD = 128
E = 320000

VP = 10240
NC = 2
NS = 16
NW = NC * NS
CH = 128
KPT = 80
EP = NW * KPT * CH
RPT = VP // NS
NSTR = 4

BLK = 256
GRID = VP // BLK


def _deg_body(dst_hbm, ones_hbm, zeros_hbm, out_hbm, dst_v, ones_v, acc):
    c = lax.axis_index("c")
    s = lax.axis_index("s")
    wid = c * NS + s
    pltpu.sync_copy(dst_hbm.at[wid], dst_v)
    pltpu.sync_copy(ones_hbm, ones_v)
    pltpu.sync_copy(zeros_hbm.at[pl.ds(s * RPT, RPT)], acc.at[pl.ds(s * RPT, RPT)])
    plsc.subcore_barrier()

    def body(j, carry):
        pltpu.sync_copy(ones_v, acc.at[dst_v.at[j]], add=True)
        return carry

    lax.fori_loop(0, KPT, body, 0)
    plsc.subcore_barrier()
    pltpu.sync_copy(acc.at[pl.ds(s * RPT, RPT)], out_hbm.at[c, pl.ds(s * RPT, RPT)])


@functools.cache
def _sc_calls():
    mesh = plsc.VectorSubcoreMesh(core_axis_name="c", subcore_axis_name="s")
    deg_call = pl.kernel(
        _deg_body,
        out_type=jax.ShapeDtypeStruct((NC, VP, HD), jnp.float32),
        mesh=mesh,
        scratch_types=[
            pltpu.VMEM((KPT, CH), jnp.int32),
            pltpu.VMEM((CH, HD), jnp.float32),
            pltpu.VMEM_SHARED((VP, HD), jnp.float32),
        ],
    )
    agg_call = pl.kernel(
        _agg_body,
        out_type=jax.ShapeDtypeStruct((NC, VP, HD), jnp.float32),
        mesh=mesh,
        scratch_types=[
            pltpu.VMEM((4, 2, CH), jnp.int32),
            pltpu.VMEM((2, CH, HD), jnp.float32),
            pltpu.VMEM_SHARED((VP, HD), jnp.float32),
        ] + [pltpu.SemaphoreType.DMA] * 8,
    )
    return deg_call, agg_call


def _agg_body(ys_hbm, src_hbm, dst_hbm, zeros_hbm, out_hbm,
              idx_v, rows, acc, g0, g1, s0, s1, i0, i1, i2, i3):
    c = lax.axis_index("c")
    s = lax.axis_index("s")
    wid = c * NS + s
    gs = (g0, g1)
    ss = (s0, s1)
    isems = (i0, i1, i2, i3)

    def fetch_idx(j, q):
        pltpu.async_copy(src_hbm.at[wid, j], idx_v.at[q, 0], isems[q])
        pltpu.async_copy(dst_hbm.at[wid, j], idx_v.at[q, 1], isems[q])

    def wait_idx(j, q):
        pltpu.make_async_copy(src_hbm.at[wid, j], idx_v.at[q, 0], isems[q]).wait()
        pltpu.make_async_copy(dst_hbm.at[wid, j], idx_v.at[q, 1], isems[q]).wait()

    def gather(q, b):
        for k in range(NSTR):
            pltpu.async_copy(
                ys_hbm.at[idx_v.at[q, 0, pl.ds(k * (CH // NSTR), CH // NSTR)]],
                rows.at[b, pl.ds(k * (CH // NSTR), CH // NSTR)], gs[b])

    def wait_gather(q, b):
        for k in range(NSTR):
            pltpu.make_async_copy(
                ys_hbm.at[idx_v.at[q, 0, pl.ds(k * (CH // NSTR), CH // NSTR)]],
                rows.at[b, pl.ds(k * (CH // NSTR), CH // NSTR)], gs[b]).wait()

    def wait_scatter(q, b):
        pltpu.make_async_copy(rows.at[b], acc.at[idx_v.at[q, 1]], ss[b]).wait()

    fetch_idx(0, 0)
    fetch_idx(1, 1)
    fetch_idx(2, 2)
    pltpu.sync_copy(zeros_hbm.at[pl.ds(s * RPT, RPT)], acc.at[pl.ds(s * RPT, RPT)])
    plsc.subcore_barrier()
    wait_idx(0, 0)
    gather(0, 0)

    def outer(i, carry):
        for k in range(4):
            j = 4 * i + k
            b = k % 2
            nb = 1 - b
            wait_gather(k, b)
            pltpu.async_copy(rows.at[b], acc.at[idx_v.at[k, 1]], ss[b], add=True)

            @pl.when(j >= 1)
            def _():
                wait_scatter((k - 1) % 4, nb)

            @pl.when(j + 3 < KPT)
            def _():
                fetch_idx(j + 3, (k + 3) % 4)

            @pl.when(j + 1 < KPT)
            def _():
                wait_idx(j + 1, (k + 1) % 4)
                gather((k + 1) % 4, nb)
        return carry

    lax.fori_loop(0, KPT // 4, outer, 0)
    wait_scatter(3, 1)
    plsc.subcore_barrier()
    pltpu.sync_copy(acc.at[pl.ds(s * RPT, RPT)], out_hbm.at[c, pl.ds(s * RPT, RPT)])


def _prep_body(lat_ref, wl_ref, b_ref, out_ref):
    out_ref[...] = (
        jnp.dot(lat_ref[...], wl_ref[...], preferred_element_type=jnp.float32, precision=lax.Precision.HIGHEST)
        + b_ref[...]
    )


def _dis_from_partials(p0_ref, p1_ref):
    p = p0_ref[...][:, 0:1] + p1_ref[...][:, 0:1]
    deg = 4.0 * p + 1.0
    return lax.rsqrt(deg)


def _in_body(xyz_ref, w3_ref, co_ref, p0_ref, p1_ref, w_ref, xw_ref, ys_ref):
    base = jnp.dot(xyz_ref[...], w3_ref[...], preferred_element_type=jnp.float32, precision=lax.Precision.HIGHEST)
    co = co_ref[...]
    W = w_ref[...]
    dis = _dis_from_partials(p0_ref, p1_ref)
    for b in range(B):
        h = jnp.maximum(base + co[b:b + 1, :], 0.0)
        xw = jnp.dot(h, W, preferred_element_type=jnp.float32, precision=lax.Precision.HIGHEST)
        xw_ref[b, :, :] = xw
        if b == 0:
            ys_ref[...] = dis * xw


def _mid_body(xw_ref, acc_ref, p0_ref, p1_ref, b_ref, wn_ref, xwn_ref, ys_ref):
    dis = _dis_from_partials(p0_ref, p1_ref)
    bias = b_ref[...][0:1, :]
    Wn = wn_ref[...]
    a = acc_ref[0, :, :] + acc_ref[1, :, :]
    g0 = jnp.maximum(4.0 * dis * a + (dis * dis) * xw_ref[0, :, :] + bias, 0.0)
    xwn0 = jnp.dot(g0, Wn, preferred_element_type=jnp.float32, precision=lax.Precision.HIGHEST)
    xwn_ref[0, :, :] = xwn0
    ys_ref[...] = dis * xwn0
    for b in range(1, B):
        g = jnp.maximum(xw_ref[b, :, :] + bias, 0.0)
        xwn_ref[b, :, :] = jnp.dot(g, Wn, preferred_element_type=jnp.float32, precision=lax.Precision.HIGHEST)


def _out_body(xw_ref, acc_ref, p0_ref, p1_ref, b_ref, wo_ref, bo_ref, o_ref):
    dis = _dis_from_partials(p0_ref, p1_ref)
    bias = b_ref[...][0:1, :]
    Wo = wo_ref[...]
    bo = bo_ref[...][0:1, :]
    a = acc_ref[0, :, :] + acc_ref[1, :, :]
    g0 = jnp.maximum(4.0 * dis * a + (dis * dis) * xw_ref[0, :, :] + bias, 0.0)
    o_ref[0, :, :] = jnp.dot(g0, Wo, preferred_element_type=jnp.float32, precision=lax.Precision.HIGHEST) + bo
    for b in range(1, B):
        g = jnp.maximum(xw_ref[b, :, :] + bias, 0.0)
        o_ref[b, :, :] = jnp.dot(g, Wo, preferred_element_type=jnp.float32, precision=lax.Precision.HIGHEST) + bo


def _row_spec(minor):
    return pl.BlockSpec((BLK, minor), lambda i: (i, 0))


def _full_spec(shape):
    nd = len(shape)
    return pl.BlockSpec(shape, lambda i, _nd=nd: (0,) * _nd)


_in_call = pl.pallas_call(
    _in_body,
    grid=(GRID,),
    in_specs=[
        _row_spec(128),
        _full_spec((128, 128)),
        _full_spec((8, 128)),
        _row_spec(128),
        _row_spec(128),
        _full_spec((128, 128)),
    ],
    out_specs=[
        pl.BlockSpec((B, BLK, 128), lambda i: (0, i, 0)),
        _row_spec(128),
    ],
    out_shape=[
        jax.ShapeDtypeStruct((B, VP, 128), jnp.float32),
        jax.ShapeDtypeStruct((VP, 128), jnp.float32),
    ],
)

_mid_call = pl.pallas_call(
    _mid_body,
    grid=(GRID,),
    in_specs=[
        pl.BlockSpec((B, BLK, 128), lambda i: (0, i, 0)),
        pl.BlockSpec((NC, BLK, 128), lambda i: (0, i, 0)),
        _row_spec(128),
        _row_spec(128),
        _full_spec((8, 128)),
        _full_spec((128, 128)),
    ],
    out_specs=[
        pl.BlockSpec((B, BLK, 128), lambda i: (0, i, 0)),
        _row_spec(128),
    ],
    out_shape=[
        jax.ShapeDtypeStruct((B, VP, 128), jnp.float32),
        jax.ShapeDtypeStruct((VP, 128), jnp.float32),
    ],
)

_fin_call = pl.pallas_call(
    _out_body,
    grid=(GRID,),
    in_specs=[
        pl.BlockSpec((B, BLK, 128), lambda i: (0, i, 0)),
        pl.BlockSpec((NC, BLK, 128), lambda i: (0, i, 0)),
        _row_spec(128),
        _row_spec(128),
        _full_spec((8, 128)),
        _full_spec((128, 128)),
        _full_spec((8, 128)),
    ],
    out_specs=pl.BlockSpec((B, BLK, 128), lambda i: (0, i, 0)),
    out_shape=jax.ShapeDtypeStruct((B, VP, 128), jnp.float32),
)

_prep_call = pl.pallas_call(
    _prep_body,
    out_shape=jax.ShapeDtypeStruct((8, 128), jnp.float32),
)


@jax.jit
def kernel(vertex_xyz, latent, edge_index, W_in, b_in,
           W_c0, b_c0, W_c1, b_c1, W_c2, b_c2, W_out, b_out):
    f32 = jnp.float32

    src = edge_index[0]
    dst = edge_index[1]
    pad = V + (jnp.arange(EP - E, dtype=jnp.int32) % (VP - V))
    src3 = jnp.concatenate([src, pad]).reshape(NW, KPT, CH)
    dst3 = jnp.concatenate([dst, pad]).reshape(NW, KPT, CH)

    xyz128 = jnp.zeros((VP, 128), f32).at[:V, :3].set(vertex_xyz)
    W3p = jnp.zeros((128, 128), f32).at[:3, :].set(W_in[:3])
    Wl = W_in[3:]
    lat8 = jnp.zeros((8, Wl.shape[0]), f32).at[:B, :].set(latent)
    bin8 = jnp.broadcast_to(b_in[None, :], (8, 128))
    bc = [jnp.broadcast_to(bb[None, :], (8, 128)) for bb in (b_c0, b_c1, b_c2)]
    Wo128 = jnp.zeros((128, 128), f32).at[:, :3].set(W_out)
    bo8 = jnp.zeros((8, 128), f32).at[0, :3].set(b_out)

    ones128 = jnp.ones((CH, HD), f32)
    zeros128 = jnp.zeros((VP, HD), f32)

    _deg_call, _agg_call = _sc_calls()
    degp = _deg_call(dst3, ones128, zeros128)
    p0 = degp[0]
    p1 = degp[1]

    consts = _prep_call(lat8, Wl, bin8)
    xw, ys = _in_call(xyz128, W3p, consts, p0, p1, W_c0)

    for l, Wn in ((0, W_c1), (1, W_c2)):
        accp = _agg_call(ys, src3, dst3, zeros128)
        xw, ys = _mid_call(xw, accp, p0, p1, bc[l], Wn)

    accp = _agg_call(ys, src3, dst3, zeros128)
    o = _fin_call(xw, accp, p0, p1, bc[2], Wo128, bo8)
    return o[:, :V, :3]


# speedup vs baseline: 2.6845x; 1.0618x over previous
import functools

import jax
import jax.numpy as jnp
from jax import lax
from jax.experimental import pallas as pl
from jax.experimental.pallas import tpu as pltpu
from jax.experimental.pallas import tpu_sc as plsc

V = 10000
B = 4
HD = 128
E = 320000

VP = 10240
NC = 2
NS = 16
NW = NC * NS
CH = 128
KPT = 80
EP = NW * KPT * CH
RPT = VP // NS
NSTR = 4

BLK = 512
GRID = VP // BLK


def _deg_body(dst_hbm, ones_hbm, zeros_hbm, out_hbm, dst_v, ones_v, acc,
              s0, s1, s2, s3):
    c = lax.axis_index("c")
    s = lax.axis_index("s")
    wid = c * NS + s
    ssems = (s0, s1, s2, s3)
    pltpu.sync_copy(dst_hbm.at[wid], dst_v)
    pltpu.sync_copy(ones_hbm, ones_v)
    pltpu.sync_copy(zeros_hbm.at[pl.ds(s * RPT, RPT)], acc.at[pl.ds(s * RPT, RPT)])
    plsc.subcore_barrier()

    def outer(i, carry):
        for k in range(4):
            j = 4 * i + k

            @pl.when(j >= 4)
            def _():
                pltpu.make_async_copy(
                    ones_v, acc.at[dst_v.at[j - 4]], ssems[k]).wait()

            pltpu.async_copy(ones_v, acc.at[dst_v.at[j]], ssems[k], add=True)
        return carry

    lax.fori_loop(0, KPT // 4, outer, 0)
    for k in range(4):
        pltpu.make_async_copy(
            ones_v, acc.at[dst_v.at[KPT - 4 + k]], ssems[k]).wait()
    plsc.subcore_barrier()
    pltpu.sync_copy(acc.at[pl.ds(s * RPT, RPT)], out_hbm.at[c, pl.ds(s * RPT, RPT)])


@functools.cache
def _sc_calls():
    mesh = plsc.VectorSubcoreMesh(core_axis_name="c", subcore_axis_name="s")
    deg_call = pl.kernel(
        _deg_body,
        out_type=jax.ShapeDtypeStruct((NC, VP, HD), jnp.float32),
        mesh=mesh,
        scratch_types=[
            pltpu.VMEM((KPT, CH), jnp.int32),
            pltpu.VMEM((CH, HD), jnp.float32),
            pltpu.VMEM_SHARED((VP, HD), jnp.float32),
        ] + [pltpu.SemaphoreType.DMA] * 4,
    )
    agg_call = pl.kernel(
        _agg_body,
        out_type=jax.ShapeDtypeStruct((NC, VP, HD), jnp.float32),
        mesh=mesh,
        scratch_types=[
            pltpu.VMEM((4, 2, CH), jnp.int32),
            pltpu.VMEM((2, CH, HD), jnp.float32),
            pltpu.VMEM_SHARED((VP, HD), jnp.float32),
        ] + [pltpu.SemaphoreType.DMA] * 8,
    )
    return deg_call, agg_call


def _agg_body(ys_hbm, src_hbm, dst_hbm, zeros_hbm, out_hbm,
              idx_v, rows, acc, g0, g1, s0, s1, i0, i1, i2, i3):
    c = lax.axis_index("c")
    s = lax.axis_index("s")
    wid = c * NS + s
    gs = (g0, g1)
    ss = (s0, s1)
    isems = (i0, i1, i2, i3)

    def fetch_idx(j, q):
        pltpu.async_copy(src_hbm.at[wid, j], idx_v.at[q, 0], isems[q])
        pltpu.async_copy(dst_hbm.at[wid, j], idx_v.at[q, 1], isems[q])

    def wait_idx(j, q):
        pltpu.make_async_copy(src_hbm.at[wid, j], idx_v.at[q, 0], isems[q]).wait()
        pltpu.make_async_copy(dst_hbm.at[wid, j], idx_v.at[q, 1], isems[q]).wait()

    def gather(q, b):
        for k in range(NSTR):
            pltpu.async_copy(
                ys_hbm.at[idx_v.at[q, 0, pl.ds(k * (CH // NSTR), CH // NSTR)]],
                rows.at[b, pl.ds(k * (CH // NSTR), CH // NSTR)], gs[b])

    def wait_gather(q, b):
        for k in range(NSTR):
            pltpu.make_async_copy(
                ys_hbm.at[idx_v.at[q, 0, pl.ds(k * (CH // NSTR), CH // NSTR)]],
                rows.at[b, pl.ds(k * (CH // NSTR), CH // NSTR)], gs[b]).wait()

    def wait_scatter(q, b):
        pltpu.make_async_copy(rows.at[b], acc.at[idx_v.at[q, 1]], ss[b]).wait()

    fetch_idx(0, 0)
    fetch_idx(1, 1)
    fetch_idx(2, 2)
    pltpu.sync_copy(zeros_hbm.at[pl.ds(s * RPT, RPT)], acc.at[pl.ds(s * RPT, RPT)])
    plsc.subcore_barrier()
    wait_idx(0, 0)
    gather(0, 0)

    def outer(i, carry):
        for k in range(4):
            j = 4 * i + k
            b = k % 2
            nb = 1 - b
            wait_gather(k, b)
            pltpu.async_copy(rows.at[b], acc.at[idx_v.at[k, 1]], ss[b], add=True)

            @pl.when(j >= 1)
            def _():
                wait_scatter((k - 1) % 4, nb)

            @pl.when(j + 3 < KPT)
            def _():
                fetch_idx(j + 3, (k + 3) % 4)

            @pl.when(j + 1 < KPT)
            def _():
                wait_idx(j + 1, (k + 1) % 4)
                gather((k + 1) % 4, nb)
        return carry

    lax.fori_loop(0, KPT // 4, outer, 0)
    wait_scatter(3, 1)
    plsc.subcore_barrier()
    pltpu.sync_copy(acc.at[pl.ds(s * RPT, RPT)], out_hbm.at[c, pl.ds(s * RPT, RPT)])


def _dis_from_partials(p0_ref, p1_ref):
    p = p0_ref[...][:, 0:1] + p1_ref[...][:, 0:1]
    deg = 4.0 * p + 1.0
    return lax.rsqrt(deg)


def _in_body(xyz_ref, w3_ref, lat_ref, wl_ref, bin_ref, p0_ref, p1_ref,
             w_ref, xw_ref, ys_ref):
    base = jnp.dot(xyz_ref[...], w3_ref[...], preferred_element_type=jnp.float32, precision=lax.Precision.HIGHEST)
    co = (
        jnp.dot(lat_ref[...], wl_ref[...], preferred_element_type=jnp.float32, precision=lax.Precision.HIGHEST)
        + bin_ref[...]
    )
    W = w_ref[...]
    dis = _dis_from_partials(p0_ref, p1_ref)
    for b in range(B):
        h = jnp.maximum(base + co[b:b + 1, :], 0.0)
        xw = jnp.dot(h, W, preferred_element_type=jnp.float32, precision=lax.Precision.HIGHEST)
        xw_ref[b, :, :] = xw
        if b == 0:
            ys_ref[...] = dis * xw


def _mid_body(xw_ref, acc_ref, p0_ref, p1_ref, b_ref, wn_ref, xwn_ref, ys_ref):
    dis = _dis_from_partials(p0_ref, p1_ref)
    bias = b_ref[...][0:1, :]
    Wn = wn_ref[...]
    a = acc_ref[0, :, :] + acc_ref[1, :, :]
    g0 = jnp.maximum(4.0 * dis * a + (dis * dis) * xw_ref[0, :, :] + bias, 0.0)
    xwn0 = jnp.dot(g0, Wn, preferred_element_type=jnp.float32, precision=lax.Precision.HIGHEST)
    xwn_ref[0, :, :] = xwn0
    ys_ref[...] = dis * xwn0
    for b in range(1, B):
        g = jnp.maximum(xw_ref[b, :, :] + bias, 0.0)
        xwn_ref[b, :, :] = jnp.dot(g, Wn, preferred_element_type=jnp.float32, precision=lax.Precision.HIGHEST)


def _out_body(xw_ref, acc_ref, p0_ref, p1_ref, b_ref, wo_ref, bo_ref, o_ref):
    dis = _dis_from_partials(p0_ref, p1_ref)
    bias = b_ref[...][0:1, :]
    Wo = wo_ref[...]
    bo = bo_ref[...][0:1, :]
    a = acc_ref[0, :, :] + acc_ref[1, :, :]
    g0 = jnp.maximum(4.0 * dis * a + (dis * dis) * xw_ref[0, :, :] + bias, 0.0)
    o_ref[0, :, :] = jnp.dot(g0, Wo, preferred_element_type=jnp.float32, precision=lax.Precision.HIGHEST) + bo
    for b in range(1, B):
        g = jnp.maximum(xw_ref[b, :, :] + bias, 0.0)
        o_ref[b, :, :] = jnp.dot(g, Wo, preferred_element_type=jnp.float32, precision=lax.Precision.HIGHEST) + bo


def _row_spec(minor):
    return pl.BlockSpec((BLK, minor), lambda i: (i, 0))


def _full_spec(shape):
    nd = len(shape)
    return pl.BlockSpec(shape, lambda i, _nd=nd: (0,) * _nd)


_in_call = pl.pallas_call(
    _in_body,
    grid=(GRID,),
    in_specs=[
        _row_spec(128),
        _full_spec((128, 128)),
        _full_spec((8, 512)),
        _full_spec((512, 128)),
        _full_spec((8, 128)),
        _row_spec(128),
        _row_spec(128),
        _full_spec((128, 128)),
    ],
    out_specs=[
        pl.BlockSpec((B, BLK, 128), lambda i: (0, i, 0)),
        _row_spec(128),
    ],
    out_shape=[
        jax.ShapeDtypeStruct((B, VP, 128), jnp.float32),
        jax.ShapeDtypeStruct((VP, 128), jnp.float32),
    ],
)

_mid_call = pl.pallas_call(
    _mid_body,
    grid=(GRID,),
    in_specs=[
        pl.BlockSpec((B, BLK, 128), lambda i: (0, i, 0)),
        pl.BlockSpec((NC, BLK, 128), lambda i: (0, i, 0)),
        _row_spec(128),
        _row_spec(128),
        _full_spec((8, 128)),
        _full_spec((128, 128)),
    ],
    out_specs=[
        pl.BlockSpec((B, BLK, 128), lambda i: (0, i, 0)),
        _row_spec(128),
    ],
    out_shape=[
        jax.ShapeDtypeStruct((B, VP, 128), jnp.float32),
        jax.ShapeDtypeStruct((VP, 128), jnp.float32),
    ],
)

_fin_call = pl.pallas_call(
    _out_body,
    grid=(GRID,),
    in_specs=[
        pl.BlockSpec((B, BLK, 128), lambda i: (0, i, 0)),
        pl.BlockSpec((NC, BLK, 128), lambda i: (0, i, 0)),
        _row_spec(128),
        _row_spec(128),
        _full_spec((8, 128)),
        _full_spec((128, 128)),
        _full_spec((8, 128)),
    ],
    out_specs=pl.BlockSpec((B, BLK, 128), lambda i: (0, i, 0)),
    out_shape=jax.ShapeDtypeStruct((B, VP, 128), jnp.float32),
)

@jax.jit
def kernel(vertex_xyz, latent, edge_index, W_in, b_in,
           W_c0, b_c0, W_c1, b_c1, W_c2, b_c2, W_out, b_out):
    f32 = jnp.float32

    src = edge_index[0]
    dst = edge_index[1]
    pad = V + (jnp.arange(EP - E, dtype=jnp.int32) % (VP - V))
    src3 = jnp.concatenate([src, pad]).reshape(NW, KPT, CH)
    dst3 = jnp.concatenate([dst, pad]).reshape(NW, KPT, CH)

    xyz128 = jnp.zeros((VP, 128), f32).at[:V, :3].set(vertex_xyz)
    W3p = jnp.zeros((128, 128), f32).at[:3, :].set(W_in[:3])
    Wl = W_in[3:]
    lat8 = jnp.zeros((8, Wl.shape[0]), f32).at[:B, :].set(latent)
    bin8 = jnp.broadcast_to(b_in[None, :], (8, 128))
    bc = [jnp.broadcast_to(bb[None, :], (8, 128)) for bb in (b_c0, b_c1, b_c2)]
    Wo128 = jnp.zeros((128, 128), f32).at[:, :3].set(W_out)
    bo8 = jnp.zeros((8, 128), f32).at[0, :3].set(b_out)

    ones128 = jnp.ones((CH, HD), f32)
    zeros128 = jnp.zeros((VP, HD), f32)

    _deg_call, _agg_call = _sc_calls()
    degp = _deg_call(dst3, ones128, zeros128)
    p0 = degp[0]
    p1 = degp[1]

    xw, ys = _in_call(xyz128, W3p, lat8, Wl, bin8, p0, p1, W_c0)

    for l, Wn in ((0, W_c1), (1, W_c2)):
        accp = _agg_call(ys, src3, dst3, zeros128)
        xw, ys = _mid_call(xw, accp, p0, p1, bc[l], Wn)

    accp = _agg_call(ys, src3, dst3, zeros128)
    o = _fin_call(xw, accp, p0, p1, bc[2], Wo128, bo8)
    return o[:, :V, :3]
